# baseline (speedup 1.0000x reference)
import functools

import numpy as np
import jax
import jax.numpy as jnp
from jax import lax
from jax.experimental import pallas as pl
from jax.experimental.pallas import tpu as pltpu
from jax.experimental.pallas import tpu_sc as plsc

H = 32
W = 32
M = 4096
IMG = 512
SIGMA = 4.0
KS = 2 * int(4.0 * SIGMA + 0.5) + 1

SC_BATCHES = 7
SC_ROWS = SC_BATCHES * H * W
_NW = 32
_CH = 8
_NACC = 8


def _build_combined_matrix() -> np.ndarray:
    scale = IMG / H
    sample_f = (np.arange(IMG) + 0.5) / scale - 0.5
    x = np.abs(sample_f[None, :] - np.arange(H)[:, None])
    w = np.maximum(0.0, 1.0 - x)
    w = w / w.sum(axis=0, keepdims=True)
    R = w.T.astype(np.float64)

    xs = np.arange(KS, dtype=np.float64) - KS // 2
    k1 = np.exp(-(xs ** 2) / (2.0 * SIGMA ** 2))
    k1 = k1 / k1.sum()
    pad = KS // 2
    idx = np.arange(-pad, IMG + pad)
    ridx = np.where(idx < 0, -idx, np.where(idx >= IMG, 2 * IMG - 2 - idx, idx))
    G = np.zeros((IMG, IMG))
    for o in range(IMG):
        for t in range(KS):
            G[o, ridx[o + t]] += k1[t]
    return (G @ R).astype(np.float32)


_C_MATRIX = _build_combined_matrix()


def _merge3(A, B):
    a1, a2, a3 = A
    b1, b2, b3 = B
    c1 = jnp.minimum(a1, b1)
    hi = jnp.maximum(a1, b1)
    lo2 = jnp.minimum(a2, b2)
    c2 = jnp.minimum(hi, lo2)
    c3 = jnp.minimum(jnp.maximum(hi, lo2),
                     jnp.minimum(jnp.maximum(a2, b2), jnp.minimum(a3, b3)))
    return c1, c2, c3


def _score_rows_tc(x_ref):
    inf = jnp.float32(np.inf)
    a1 = x_ref[:, 0:128]
    a2 = jnp.full_like(a1, inf)
    a3 = a2
    for j in range(1, M // 128):
        v = x_ref[:, j * 128:(j + 1) * 128]
        t = jnp.maximum(a1, v)
        a1 = jnp.minimum(a1, v)
        a3 = jnp.minimum(a3, jnp.maximum(a2, t))
        a2 = jnp.minimum(a2, t)

    x = jnp.concatenate([a1, a2, a3], axis=1)
    m1 = jnp.min(x, axis=1, keepdims=True)
    eq1 = x == m1
    c1 = jnp.sum(eq1.astype(jnp.float32), axis=1, keepdims=True)
    y = jnp.where(eq1, inf, x)
    my = jnp.min(y, axis=1, keepdims=True)
    eqy = y == my
    cy = jnp.sum(eqy.astype(jnp.float32), axis=1, keepdims=True)
    z = jnp.where(eqy, inf, y)
    mz = jnp.min(z, axis=1, keepdims=True)

    m2 = jnp.where(c1 >= 2.0, m1, my)
    m3 = jnp.where(c1 >= 3.0, m1,
                   jnp.where(c1 == 2.0, my,
                             jnp.where(cy >= 2.0, my, mz)))

    d1 = jnp.sqrt(m1)
    d2 = jnp.sqrt(m2)
    d3 = jnp.sqrt(m3)
    denom = 1.0 + jnp.exp(d1 - d2) + jnp.exp(d1 - d3)
    return d1 / denom


def _apply_tail(s, c):
    t = lax.dot_general(c, s, (((1,), (0,)), ((), ())),
                        preferred_element_type=jnp.float32)
    return lax.dot_general(t, c, (((1,), (1,)), ((), ())),
                           preferred_element_type=jnp.float32)


def _fused_block_tc(x_ref, c_ref, out_ref):
    score = _score_rows_tc(x_ref)
    s = score.reshape(H, W)
    out_ref[0] = _apply_tail(s, c_ref[...])


def _tail_block(s_ref, c_ref, amap_in_ref, out_ref):
    del amap_in_ref
    out_ref[0] = _apply_tail(s_ref[0], c_ref[...])


def _vsqrt(x):
    xs = jnp.maximum(x, jnp.float32(1e-30))
    i = lax.bitcast_convert_type(xs, jnp.int32)
    i = jnp.int32(0x5F3759DF) - lax.shift_right_logical(i, 1)
    y = lax.bitcast_convert_type(i, jnp.float32)
    half = jnp.float32(0.5) * xs
    for _ in range(4):
        y = y * (jnp.float32(1.5) - half * y * y)
    return x * y


def _xlane_top3(A):
    lanes = lax.iota(jnp.int32, 16)
    dn = lax.GatherDimensionNumbers(offset_dims=(), collapsed_slice_dims=(0,),
                                    start_index_map=(0,))
    for sh in (8, 4, 2, 1):
        idx = lax.bitwise_xor(lanes, sh)
        B = tuple(
            lax.gather(v, idx[:, None], dn, slice_sizes=(1,),
                       mode=lax.GatherScatterMode.PROMISE_IN_BOUNDS)
            for v in A)
        A = _merge3(A, B)
    return A


def _row_score_sc(buf, r):
    inf = jnp.full((16,), jnp.float32(np.inf))
    accs = []
    for a in range(_NACC):
        v = buf[r, pl.ds(a * 16, 16)]
        accs.append((v, inf, inf))

    def step(j, carry):
        out = list(carry)
        for jj in range(2):
            cur = out
            out = []
            for a in range(_NACC):
                v = buf[r, pl.ds(((2 * j + jj) * _NACC + a) * 16, 16)]
                a1, a2, a3 = cur[a]
                t = jnp.maximum(a1, v)
                a1 = jnp.minimum(a1, v)
                a3 = jnp.minimum(a3, jnp.maximum(a2, t))
                a2 = jnp.minimum(a2, t)
                out.append((a1, a2, a3))
        return tuple(out)

    accs = list(lax.fori_loop(1, M // (2 * 16 * _NACC) , step, tuple(accs),
                              unroll=1))
    cur = accs
    accs = []
    for a in range(_NACC):
        v = buf[r, pl.ds((_NACC + a) * 16, 16)]
        a1, a2, a3 = cur[a]
        t = jnp.maximum(a1, v)
        a1 = jnp.minimum(a1, v)
        a3 = jnp.minimum(a3, jnp.maximum(a2, t))
        a2 = jnp.minimum(a2, t)
        accs.append((a1, a2, a3))
    while len(accs) > 1:
        accs = [_merge3(accs[i], accs[i + 1]) for i in range(0, len(accs), 2)]
    m1, m2, m3 = _xlane_top3(accs[0])

    d1 = _vsqrt(m1)
    d2 = _vsqrt(m2)
    d3 = _vsqrt(m3)
    denom = 1.0 + jnp.exp(d1 - d2) + jnp.exp(d1 - d3)
    return d1 / denom


def _make_sc_score(base_row: int, nrows: int):
    rpw = nrows // _NW
    ngrp = rpw // (2 * _CH)
    mesh = plsc.VectorSubcoreMesh(core_axis_name="c", subcore_axis_name="s")

    @functools.partial(
        pl.kernel,
        out_type=jax.ShapeDtypeStruct((nrows,), jnp.float32),
        mesh=mesh,
        scratch_types=[
            pltpu.VMEM((_CH, M), jnp.float32),
            pltpu.VMEM((_CH, M), jnp.float32),
            pltpu.VMEM((rpw,), jnp.float32),
            pltpu.SemaphoreType.DMA,
            pltpu.SemaphoreType.DMA,
        ],
    )
    def sc_score(dist_hbm, out_hbm, buf0, buf1, sbuf, sem0, sem1):
        wid = lax.axis_index("s") * 2 + lax.axis_index("c")
        out_base = wid * rpw
        base = base_row + out_base
        bufs = (buf0, buf1)
        sems = (sem0, sem1)
        for b in range(2):
            pltpu.make_async_copy(
                dist_hbm.at[pl.ds(base + b * _CH, _CH)], bufs[b], sems[b]
            ).start()

        lanes = lax.iota(jnp.int32, 16)

        def group(g, carry):
            vec = jnp.zeros((16,), jnp.float32)
            for b in range(2):
                row0 = base + (2 * g + b) * _CH
                pltpu.make_async_copy(
                    dist_hbm.at[pl.ds(row0, _CH)], bufs[b], sems[b]
                ).wait()
                for r in range(_CH):
                    s = _row_score_sc(bufs[b], r)
                    vec = jnp.where(lanes == b * _CH + r, s, vec)

                def _prefetch(row0=row0, b=b):
                    pltpu.make_async_copy(
                        dist_hbm.at[pl.ds(row0 + 2 * _CH, _CH)], bufs[b], sems[b]
                    ).start()
                pl.when(g < ngrp - 1)(_prefetch)
            sbuf[pl.ds(g * 16, 16)] = vec
            return carry

        lax.fori_loop(0, ngrp, group, 0)
        pltpu.sync_copy(sbuf, out_hbm.at[pl.ds(out_base, rpw)])

    return sc_score


@jax.jit
def kernel(distance, scale):
    b = distance.shape[0]
    n = b * H * W
    flat = distance.reshape(n, M)
    cmat = jnp.asarray(_C_MATRIX)

    tc_b = b - SC_BATCHES
    tc_rows = n - SC_ROWS

    score_sc = _make_sc_score(tc_rows, SC_ROWS)(flat)

    amap = pl.pallas_call(
        _fused_block_tc,
        grid=(tc_b,),
        in_specs=[
            pl.BlockSpec((H * W, M), lambda i: (i, 0)),
            pl.BlockSpec((IMG, H), lambda i: (0, 0)),
        ],
        out_specs=pl.BlockSpec((1, IMG, IMG), lambda i: (i, 0, 0)),
        out_shape=jax.ShapeDtypeStruct((b, IMG, IMG), jnp.float32),
    )(flat, cmat)

    amap = pl.pallas_call(
        _tail_block,
        grid=(SC_BATCHES,),
        in_specs=[
            pl.BlockSpec((1, H, W), lambda i: (i, 0, 0)),
            pl.BlockSpec((IMG, H), lambda i: (0, 0)),
            pl.BlockSpec(memory_space=pltpu.MemorySpace.HBM),
        ],
        out_specs=pl.BlockSpec((1, IMG, IMG), lambda i: (i + tc_b, 0, 0)),
        out_shape=jax.ShapeDtypeStruct((b, IMG, IMG), jnp.float32),
        input_output_aliases={2: 0},
    )(score_sc.reshape(SC_BATCHES, H, W), cmat, amap)

    del scale
    return amap.reshape(b, 1, IMG, IMG)


# speedup vs baseline: 1.0821x; 1.0821x over previous
import functools

import numpy as np
import jax
import jax.numpy as jnp
from jax import lax
from jax.experimental import pallas as pl
from jax.experimental.pallas import tpu as pltpu
from jax.experimental.pallas import tpu_sc as plsc

H = 32
W = 32
M = 4096
IMG = 512
SIGMA = 4.0
KS = 2 * int(4.0 * SIGMA + 0.5) + 1

SC_BATCHES = 6
SC_ROWS = SC_BATCHES * H * W
_NW = 32
_CH = 8
_NACC = 8


def _build_combined_matrix() -> np.ndarray:
    scale = IMG / H
    sample_f = (np.arange(IMG) + 0.5) / scale - 0.5
    x = np.abs(sample_f[None, :] - np.arange(H)[:, None])
    w = np.maximum(0.0, 1.0 - x)
    w = w / w.sum(axis=0, keepdims=True)
    R = w.T.astype(np.float64)

    xs = np.arange(KS, dtype=np.float64) - KS // 2
    k1 = np.exp(-(xs ** 2) / (2.0 * SIGMA ** 2))
    k1 = k1 / k1.sum()
    pad = KS // 2
    idx = np.arange(-pad, IMG + pad)
    ridx = np.where(idx < 0, -idx, np.where(idx >= IMG, 2 * IMG - 2 - idx, idx))
    G = np.zeros((IMG, IMG))
    for o in range(IMG):
        for t in range(KS):
            G[o, ridx[o + t]] += k1[t]
    return (G @ R).astype(np.float32)


_C_MATRIX = _build_combined_matrix()


def _merge3(A, B):
    a1, a2, a3 = A
    b1, b2, b3 = B
    c1 = jnp.minimum(a1, b1)
    hi = jnp.maximum(a1, b1)
    lo2 = jnp.minimum(a2, b2)
    c2 = jnp.minimum(hi, lo2)
    c3 = jnp.minimum(jnp.maximum(hi, lo2),
                     jnp.minimum(jnp.maximum(a2, b2), jnp.minimum(a3, b3)))
    return c1, c2, c3


def _score_rows_tc(x_ref):
    inf = jnp.float32(np.inf)
    a1 = x_ref[:, 0:128]
    a2 = jnp.full_like(a1, inf)
    a3 = a2
    for j in range(1, M // 128):
        v = x_ref[:, j * 128:(j + 1) * 128]
        t = jnp.maximum(a1, v)
        a1 = jnp.minimum(a1, v)
        a3 = jnp.minimum(a3, jnp.maximum(a2, t))
        a2 = jnp.minimum(a2, t)

    x = jnp.concatenate([a1, a2, a3], axis=1)
    m1 = jnp.min(x, axis=1, keepdims=True)
    eq1 = x == m1
    c1 = jnp.sum(eq1.astype(jnp.float32), axis=1, keepdims=True)
    y = jnp.where(eq1, inf, x)
    my = jnp.min(y, axis=1, keepdims=True)
    eqy = y == my
    cy = jnp.sum(eqy.astype(jnp.float32), axis=1, keepdims=True)
    z = jnp.where(eqy, inf, y)
    mz = jnp.min(z, axis=1, keepdims=True)

    m2 = jnp.where(c1 >= 2.0, m1, my)
    m3 = jnp.where(c1 >= 3.0, m1,
                   jnp.where(c1 == 2.0, my,
                             jnp.where(cy >= 2.0, my, mz)))

    d1 = jnp.sqrt(m1)
    d2 = jnp.sqrt(m2)
    d3 = jnp.sqrt(m3)
    denom = 1.0 + jnp.exp(d1 - d2) + jnp.exp(d1 - d3)
    return d1 / denom


def _apply_tail(s, c):
    t = lax.dot_general(c, s, (((1,), (0,)), ((), ())),
                        preferred_element_type=jnp.float32)
    return lax.dot_general(t, c, (((1,), (1,)), ((), ())),
                           preferred_element_type=jnp.float32)


def _fused_block_tc(x_ref, c_ref, out_ref):
    score = _score_rows_tc(x_ref)
    s = score.reshape(H, W)
    out_ref[0] = _apply_tail(s, c_ref[...])


def _tail_block(s_ref, c_ref, amap_in_ref, out_ref):
    del amap_in_ref
    out_ref[0] = _apply_tail(s_ref[0], c_ref[...])


def _vsqrt(x):
    xs = jnp.maximum(x, jnp.float32(1e-30))
    i = lax.bitcast_convert_type(xs, jnp.int32)
    i = jnp.int32(0x5F3759DF) - lax.shift_right_logical(i, 1)
    y = lax.bitcast_convert_type(i, jnp.float32)
    half = jnp.float32(0.5) * xs
    for _ in range(4):
        y = y * (jnp.float32(1.5) - half * y * y)
    return x * y


def _xlane_top3(A):
    lanes = lax.iota(jnp.int32, 16)
    dn = lax.GatherDimensionNumbers(offset_dims=(), collapsed_slice_dims=(0,),
                                    start_index_map=(0,))
    for sh in (8, 4, 2, 1):
        idx = lax.bitwise_xor(lanes, sh)
        B = tuple(
            lax.gather(v, idx[:, None], dn, slice_sizes=(1,),
                       mode=lax.GatherScatterMode.PROMISE_IN_BOUNDS)
            for v in A)
        A = _merge3(A, B)
    return A


def _row_score_sc(buf, r):
    inf = jnp.full((16,), jnp.float32(np.inf))
    accs = []
    for a in range(_NACC):
        v = buf[r, pl.ds(a * 16, 16)]
        accs.append((v, inf, inf))

    def step(j, carry):
        out = []
        for a in range(_NACC):
            v = buf[r, pl.ds((j * _NACC + a) * 16, 16)]
            a1, a2, a3 = carry[a]
            t = jnp.maximum(a1, v)
            a1 = jnp.minimum(a1, v)
            a3 = jnp.minimum(a3, jnp.maximum(a2, t))
            a2 = jnp.minimum(a2, t)
            out.append((a1, a2, a3))
        return tuple(out)

    accs = list(lax.fori_loop(1, M // (16 * _NACC), step, tuple(accs)))
    while len(accs) > 1:
        accs = [_merge3(accs[i], accs[i + 1]) for i in range(0, len(accs), 2)]
    m1, m2, m3 = _xlane_top3(accs[0])

    d1 = _vsqrt(m1)
    d2 = _vsqrt(m2)
    d3 = _vsqrt(m3)
    denom = 1.0 + jnp.exp(d1 - d2) + jnp.exp(d1 - d3)
    return d1 / denom


def _make_sc_score(base_row: int, nrows: int):
    rpw = nrows // _NW
    ngrp = rpw // (2 * _CH)
    mesh = plsc.VectorSubcoreMesh(core_axis_name="c", subcore_axis_name="s")

    @functools.partial(
        pl.kernel,
        out_type=jax.ShapeDtypeStruct((nrows,), jnp.float32),
        mesh=mesh,
        scratch_types=[
            pltpu.VMEM((_CH, M), jnp.float32),
            pltpu.VMEM((_CH, M), jnp.float32),
            pltpu.VMEM((rpw,), jnp.float32),
            pltpu.SemaphoreType.DMA,
            pltpu.SemaphoreType.DMA,
        ],
    )
    def sc_score(dist_hbm, out_hbm, buf0, buf1, sbuf, sem0, sem1):
        wid = lax.axis_index("s") * 2 + lax.axis_index("c")
        out_base = wid * rpw
        base = base_row + out_base
        bufs = (buf0, buf1)
        sems = (sem0, sem1)
        for b in range(2):
            pltpu.make_async_copy(
                dist_hbm.at[pl.ds(base + b * _CH, _CH)], bufs[b], sems[b]
            ).start()

        lanes = lax.iota(jnp.int32, 16)

        def group(g, carry):
            vec = jnp.zeros((16,), jnp.float32)
            for b in range(2):
                row0 = base + (2 * g + b) * _CH
                pltpu.make_async_copy(
                    dist_hbm.at[pl.ds(row0, _CH)], bufs[b], sems[b]
                ).wait()
                for r in range(_CH):
                    s = _row_score_sc(bufs[b], r)
                    vec = jnp.where(lanes == b * _CH + r, s, vec)

                def _prefetch(row0=row0, b=b):
                    pltpu.make_async_copy(
                        dist_hbm.at[pl.ds(row0 + 2 * _CH, _CH)], bufs[b], sems[b]
                    ).start()
                pl.when(g < ngrp - 1)(_prefetch)
            sbuf[pl.ds(g * 16, 16)] = vec
            return carry

        lax.fori_loop(0, ngrp, group, 0)
        pltpu.sync_copy(sbuf, out_hbm.at[pl.ds(out_base, rpw)])

    return sc_score


@jax.jit
def kernel(distance, scale):
    b = distance.shape[0]
    n = b * H * W
    flat = distance.reshape(n, M)
    cmat = jnp.asarray(_C_MATRIX)

    tc_b = b - SC_BATCHES
    tc_rows = n - SC_ROWS

    score_sc = _make_sc_score(tc_rows, SC_ROWS)(flat)

    amap = pl.pallas_call(
        _fused_block_tc,
        grid=(tc_b,),
        in_specs=[
            pl.BlockSpec((H * W, M), lambda i: (i, 0)),
            pl.BlockSpec((IMG, H), lambda i: (0, 0)),
        ],
        out_specs=pl.BlockSpec((1, IMG, IMG), lambda i: (i, 0, 0)),
        out_shape=jax.ShapeDtypeStruct((b, IMG, IMG), jnp.float32),
    )(flat, cmat)

    amap = pl.pallas_call(
        _tail_block,
        grid=(SC_BATCHES,),
        in_specs=[
            pl.BlockSpec((1, H, W), lambda i: (i, 0, 0)),
            pl.BlockSpec((IMG, H), lambda i: (0, 0)),
            pl.BlockSpec(memory_space=pltpu.MemorySpace.HBM),
        ],
        out_specs=pl.BlockSpec((1, IMG, IMG), lambda i: (i + tc_b, 0, 0)),
        out_shape=jax.ShapeDtypeStruct((b, IMG, IMG), jnp.float32),
        input_output_aliases={2: 0},
    )(score_sc.reshape(SC_BATCHES, H, W), cmat, amap)

    del scale
    return amap.reshape(b, 1, IMG, IMG)


# speedup vs baseline: 1.0850x; 1.0027x over previous
import functools

import numpy as np
import jax
import jax.numpy as jnp
from jax import lax
from jax.experimental import pallas as pl
from jax.experimental.pallas import tpu as pltpu
from jax.experimental.pallas import tpu_sc as plsc

H = 32
W = 32
M = 4096
IMG = 512
SIGMA = 4.0
KS = 2 * int(4.0 * SIGMA + 0.5) + 1

SC_BATCHES = 5
SC_ROWS = SC_BATCHES * H * W
_NW = 32
_CH = 8
_NACC = 8


def _build_combined_matrix() -> np.ndarray:
    scale = IMG / H
    sample_f = (np.arange(IMG) + 0.5) / scale - 0.5
    x = np.abs(sample_f[None, :] - np.arange(H)[:, None])
    w = np.maximum(0.0, 1.0 - x)
    w = w / w.sum(axis=0, keepdims=True)
    R = w.T.astype(np.float64)

    xs = np.arange(KS, dtype=np.float64) - KS // 2
    k1 = np.exp(-(xs ** 2) / (2.0 * SIGMA ** 2))
    k1 = k1 / k1.sum()
    pad = KS // 2
    idx = np.arange(-pad, IMG + pad)
    ridx = np.where(idx < 0, -idx, np.where(idx >= IMG, 2 * IMG - 2 - idx, idx))
    G = np.zeros((IMG, IMG))
    for o in range(IMG):
        for t in range(KS):
            G[o, ridx[o + t]] += k1[t]
    return (G @ R).astype(np.float32)


_C_MATRIX = _build_combined_matrix()


def _merge3(A, B):
    a1, a2, a3 = A
    b1, b2, b3 = B
    c1 = jnp.minimum(a1, b1)
    hi = jnp.maximum(a1, b1)
    lo2 = jnp.minimum(a2, b2)
    c2 = jnp.minimum(hi, lo2)
    c3 = jnp.minimum(jnp.maximum(hi, lo2),
                     jnp.minimum(jnp.maximum(a2, b2), jnp.minimum(a3, b3)))
    return c1, c2, c3


def _score_rows_tc(x_ref):
    inf = jnp.float32(np.inf)
    a1 = x_ref[:, 0:128]
    a2 = jnp.full_like(a1, inf)
    a3 = a2
    for j in range(1, M // 128):
        v = x_ref[:, j * 128:(j + 1) * 128]
        t = jnp.maximum(a1, v)
        a1 = jnp.minimum(a1, v)
        a3 = jnp.minimum(a3, jnp.maximum(a2, t))
        a2 = jnp.minimum(a2, t)

    x = jnp.concatenate([a1, a2, a3], axis=1)
    m1 = jnp.min(x, axis=1, keepdims=True)
    eq1 = x == m1
    c1 = jnp.sum(eq1.astype(jnp.float32), axis=1, keepdims=True)
    y = jnp.where(eq1, inf, x)
    my = jnp.min(y, axis=1, keepdims=True)
    eqy = y == my
    cy = jnp.sum(eqy.astype(jnp.float32), axis=1, keepdims=True)
    z = jnp.where(eqy, inf, y)
    mz = jnp.min(z, axis=1, keepdims=True)

    m2 = jnp.where(c1 >= 2.0, m1, my)
    m3 = jnp.where(c1 >= 3.0, m1,
                   jnp.where(c1 == 2.0, my,
                             jnp.where(cy >= 2.0, my, mz)))

    d1 = jnp.sqrt(m1)
    d2 = jnp.sqrt(m2)
    d3 = jnp.sqrt(m3)
    denom = 1.0 + jnp.exp(d1 - d2) + jnp.exp(d1 - d3)
    return d1 / denom


def _apply_tail(s, c):
    t = lax.dot_general(c, s, (((1,), (0,)), ((), ())),
                        preferred_element_type=jnp.float32)
    return lax.dot_general(t, c, (((1,), (1,)), ((), ())),
                           preferred_element_type=jnp.float32)


def _fused_block_tc(x_ref, c_ref, out_ref):
    score = _score_rows_tc(x_ref)
    s = score.reshape(H, W)
    out_ref[0] = _apply_tail(s, c_ref[...])


def _tail_block(s_ref, c_ref, amap_in_ref, out_ref):
    del amap_in_ref
    out_ref[0] = _apply_tail(s_ref[0], c_ref[...])


def _vsqrt(x):
    xs = jnp.maximum(x, jnp.float32(1e-30))
    i = lax.bitcast_convert_type(xs, jnp.int32)
    i = jnp.int32(0x5F3759DF) - lax.shift_right_logical(i, 1)
    y = lax.bitcast_convert_type(i, jnp.float32)
    half = jnp.float32(0.5) * xs
    for _ in range(4):
        y = y * (jnp.float32(1.5) - half * y * y)
    return x * y


def _xlane_top3(A):
    lanes = lax.iota(jnp.int32, 16)
    dn = lax.GatherDimensionNumbers(offset_dims=(), collapsed_slice_dims=(0,),
                                    start_index_map=(0,))
    for sh in (8, 4, 2, 1):
        idx = lax.bitwise_xor(lanes, sh)
        B = tuple(
            lax.gather(v, idx[:, None], dn, slice_sizes=(1,),
                       mode=lax.GatherScatterMode.PROMISE_IN_BOUNDS)
            for v in A)
        A = _merge3(A, B)
    return A


def _row_score_sc(buf, r):
    inf = jnp.full((16,), jnp.float32(np.inf))
    accs = []
    for a in range(_NACC):
        v = buf[r, pl.ds(a * 16, 16)]
        accs.append((v, inf, inf))

    def step(j, carry):
        out = []
        for a in range(_NACC):
            v = buf[r, pl.ds((j * _NACC + a) * 16, 16)]
            a1, a2, a3 = carry[a]
            t = jnp.maximum(a1, v)
            a1 = jnp.minimum(a1, v)
            a3 = jnp.minimum(a3, jnp.maximum(a2, t))
            a2 = jnp.minimum(a2, t)
            out.append((a1, a2, a3))
        return tuple(out)

    accs = list(lax.fori_loop(1, M // (16 * _NACC), step, tuple(accs)))
    while len(accs) > 1:
        accs = [_merge3(accs[i], accs[i + 1]) for i in range(0, len(accs), 2)]
    m1, m2, m3 = _xlane_top3(accs[0])

    d1 = _vsqrt(m1)
    d2 = _vsqrt(m2)
    d3 = _vsqrt(m3)
    denom = 1.0 + jnp.exp(d1 - d2) + jnp.exp(d1 - d3)
    return d1 / denom


def _make_sc_score(base_row: int, nrows: int):
    rpw = nrows // _NW
    ngrp = rpw // (2 * _CH)
    mesh = plsc.VectorSubcoreMesh(core_axis_name="c", subcore_axis_name="s")

    @functools.partial(
        pl.kernel,
        out_type=jax.ShapeDtypeStruct((nrows,), jnp.float32),
        mesh=mesh,
        scratch_types=[
            pltpu.VMEM((_CH, M), jnp.float32),
            pltpu.VMEM((_CH, M), jnp.float32),
            pltpu.VMEM((rpw,), jnp.float32),
            pltpu.SemaphoreType.DMA,
            pltpu.SemaphoreType.DMA,
        ],
    )
    def sc_score(dist_hbm, out_hbm, buf0, buf1, sbuf, sem0, sem1):
        wid = lax.axis_index("s") * 2 + lax.axis_index("c")
        out_base = wid * rpw
        base = base_row + out_base
        bufs = (buf0, buf1)
        sems = (sem0, sem1)
        for b in range(2):
            pltpu.make_async_copy(
                dist_hbm.at[pl.ds(base + b * _CH, _CH)], bufs[b], sems[b]
            ).start()

        lanes = lax.iota(jnp.int32, 16)

        def group(g, carry):
            vec = jnp.zeros((16,), jnp.float32)
            for b in range(2):
                row0 = base + (2 * g + b) * _CH
                pltpu.make_async_copy(
                    dist_hbm.at[pl.ds(row0, _CH)], bufs[b], sems[b]
                ).wait()
                for r in range(_CH):
                    s = _row_score_sc(bufs[b], r)
                    vec = jnp.where(lanes == b * _CH + r, s, vec)

                def _prefetch(row0=row0, b=b):
                    pltpu.make_async_copy(
                        dist_hbm.at[pl.ds(row0 + 2 * _CH, _CH)], bufs[b], sems[b]
                    ).start()
                pl.when(g < ngrp - 1)(_prefetch)
            sbuf[pl.ds(g * 16, 16)] = vec
            return carry

        lax.fori_loop(0, ngrp, group, 0)
        pltpu.sync_copy(sbuf, out_hbm.at[pl.ds(out_base, rpw)])

    return sc_score


@jax.jit
def kernel(distance, scale):
    b = distance.shape[0]
    n = b * H * W
    flat = distance.reshape(n, M)
    cmat = jnp.asarray(_C_MATRIX)

    tc_b = b - SC_BATCHES
    tc_rows = n - SC_ROWS

    score_sc = _make_sc_score(tc_rows, SC_ROWS)(flat)

    amap = pl.pallas_call(
        _fused_block_tc,
        grid=(tc_b,),
        in_specs=[
            pl.BlockSpec((H * W, M), lambda i: (i, 0)),
            pl.BlockSpec((IMG, H), lambda i: (0, 0)),
        ],
        out_specs=pl.BlockSpec((1, IMG, IMG), lambda i: (i, 0, 0)),
        out_shape=jax.ShapeDtypeStruct((b, IMG, IMG), jnp.float32),
    )(flat, cmat)

    amap = pl.pallas_call(
        _tail_block,
        grid=(SC_BATCHES,),
        in_specs=[
            pl.BlockSpec((1, H, W), lambda i: (i, 0, 0)),
            pl.BlockSpec((IMG, H), lambda i: (0, 0)),
            pl.BlockSpec(memory_space=pltpu.MemorySpace.HBM),
        ],
        out_specs=pl.BlockSpec((1, IMG, IMG), lambda i: (i + tc_b, 0, 0)),
        out_shape=jax.ShapeDtypeStruct((b, IMG, IMG), jnp.float32),
        input_output_aliases={2: 0},
    )(score_sc.reshape(SC_BATCHES, H, W), cmat, amap)

    del scale
    return amap.reshape(b, 1, IMG, IMG)
